# baseline (speedup 1.0000x reference)
import functools

import jax
import jax.numpy as jnp
from jax import lax
from jax.experimental import pallas as pl
from jax.experimental.pallas import tpu as pltpu
from jax.experimental.pallas import tpu_sc as plsc

N = 10000
NPAD = 10240
E = 320000
D_IN = 128
D_HID = 16
PK = 128 // D_HID
NP_P = N // PK
NPAD_P = NPAD // PK

NC = 2
NS = 16
NW = NC * NS
EPW = E // NW
CHUNKS = 25
CHUNK = EPW // CHUNKS
NBUF = 4
NODES_PER_S = NPAD // NS

_mesh = plsc.VectorSubcoreMesh(core_axis_name="c", subcore_axis_name="s")


@functools.partial(
    pl.kernel,
    out_type=jax.ShapeDtypeStruct((NC, NPAD), jnp.float32),
    mesh=_mesh,
    scratch_types=[
        [pltpu.VMEM((CHUNK,), jnp.int32) for _ in range(CHUNKS)],
        pltpu.VMEM((CHUNK,), jnp.float32),
        pltpu.VMEM_SHARED((NPAD,), jnp.float32),
        pltpu.SemaphoreType.DMA,
        pltpu.SemaphoreType.DMA,
    ],
    compiler_params=pltpu.CompilerParams(use_tc_tiling_on_sc=False),
)
def _deg_kernel(ei_hbm, ones_hbm, zeros_hbm, out_hbm, dst_vs, ones_v, acc_s,
                isem, ssem):
    c = lax.axis_index("c")
    s = lax.axis_index("s")
    wid = s * NC + c
    sl = pl.ds(s * NODES_PER_S, NODES_PER_S)
    e0 = wid * EPW
    cps = [pltpu.async_copy(ei_hbm.at[1, pl.ds(e0 + k * CHUNK, CHUNK)],
                            dst_vs[k], isem) for k in range(CHUNKS)]
    one_cp = pltpu.async_copy(ones_hbm, ones_v, isem)
    pltpu.sync_copy(zeros_hbm.at[sl], acc_s.at[sl])
    for cp in cps:
        cp.wait()
    one_cp.wait()
    plsc.subcore_barrier()
    scps = [pltpu.async_copy(ones_v, acc_s.at[dst_vs[k]], ssem, add=True)
            for k in range(CHUNKS)]
    for cp in scps:
        cp.wait()
    plsc.subcore_barrier()
    pltpu.sync_copy(acc_s.at[sl], out_hbm.at[c, sl])


@functools.partial(
    pl.kernel,
    out_type=jax.ShapeDtypeStruct((NC, NPAD, D_HID), jnp.float32),
    mesh=_mesh,
    scratch_types=[
        pltpu.VMEM((EPW,), jnp.int32),
        [pltpu.VMEM((CHUNK,), jnp.int32) for _ in range(CHUNKS)],
        [pltpu.VMEM((CHUNK, D_HID), jnp.float32) for _ in range(NBUF)],
        pltpu.VMEM_SHARED((NPAD, D_HID), jnp.float32),
        pltpu.SemaphoreType.DMA,
        [pltpu.SemaphoreType.DMA for _ in range(NBUF)],
        [pltpu.SemaphoreType.DMA for _ in range(NBUF)],
    ],
    compiler_params=pltpu.CompilerParams(use_tc_tiling_on_sc=False),
)
def _agg_kernel(ei_hbm, hs_hbm, zeros_hbm, out_hbm,
                src_all, dst_vs, rows_vs, acc_s, isem, gsems, ssems):
    c = lax.axis_index("c")
    s = lax.axis_index("s")
    wid = s * NC + c
    sl = pl.ds(s * NODES_PER_S, NODES_PER_S)
    e0 = wid * EPW

    def gather(k):
        return pltpu.async_copy(
            hs_hbm.at[src_all.at[pl.ds(k * CHUNK, CHUNK)]],
            rows_vs[k % NBUF], gsems[k % NBUF])

    src_cp = pltpu.async_copy(ei_hbm.at[0, pl.ds(e0, EPW)], src_all, isem)
    dst_cps = [pltpu.async_copy(ei_hbm.at[1, pl.ds(e0 + k * CHUNK, CHUNK)],
                                dst_vs[k], isem) for k in range(CHUNKS)]
    pltpu.sync_copy(zeros_hbm.at[sl], acc_s.at[sl])
    src_cp.wait()
    for cp in dst_cps:
        cp.wait()
    plsc.subcore_barrier()
    gcps = [None] * NBUF
    scps = [None] * NBUF
    for k in range(min(NBUF, CHUNKS)):
        gcps[k] = gather(k)
    for k in range(CHUNKS):
        b = k % NBUF
        gcps[b].wait()
        scps[b] = pltpu.async_copy(rows_vs[b], acc_s.at[dst_vs[k]], ssems[b],
                                   add=True)
        if k + NBUF < CHUNKS:
            scps[b].wait()
            gcps[b] = gather(k + NBUF)
    for k in range(max(0, CHUNKS - NBUF), CHUNKS):
        scps[k % NBUF].wait()
    plsc.subcore_barrier()
    pltpu.sync_copy(acc_s.at[sl], out_hbm.at[c, sl])


def _tcmm_body(xp_ref, w1b_ref, h_ref):
    h_ref[:NP_P, :] = jnp.dot(xp_ref[...], w1b_ref[...],
                              preferred_element_type=jnp.float32)
    h_ref[NP_P:, :] = jnp.zeros((NPAD_P - NP_P, 128), jnp.float32)


def _tcscale_body(degp_ref, exp_ref, h_ref, hs_ref, dis_ref):
    g = lax.rsqrt(1.0 + degp_ref[0] + degp_ref[1])
    dis = jnp.dot(g, exp_ref[...], preferred_element_type=jnp.float32)
    hs_ref[...] = dis * h_ref[...]
    dis_ref[...] = dis


def _tc2_body(aggp_ref, hs1_ref, dis_ref, b1_ref, w2b_ref, hs2_ref):
    dis = dis_ref[...]
    t = dis * (aggp_ref[0] + aggp_ref[1] + hs1_ref[...])
    t = jnp.maximum(t + b1_ref[...], 0.0)
    u = jnp.dot(t, w2b_ref[...], preferred_element_type=jnp.float32)
    hs2_ref[:NP_P, :] = dis[:NP_P, :] * u[:NP_P, :]
    hs2_ref[NP_P:, :] = jnp.zeros((NPAD_P - NP_P, 128), jnp.float32)


def _tc3_body(aggp_ref, hs2_ref, dis_ref, b2_ref, out_ref):
    v = dis_ref[:NP_P, :] * (aggp_ref[0, :NP_P, :] + aggp_ref[1, :NP_P, :]
                             + hs2_ref[:NP_P, :]) + b2_ref[...]
    out_ref[...] = jax.nn.sigmoid(v)


_tcmm = pl.pallas_call(
    _tcmm_body,
    out_shape=jax.ShapeDtypeStruct((NPAD_P, 128), jnp.float32),
)

_tcscale = pl.pallas_call(
    _tcscale_body,
    out_shape=[jax.ShapeDtypeStruct((NPAD_P, 128), jnp.float32),
               jax.ShapeDtypeStruct((NPAD_P, 128), jnp.float32)],
)

_tc2 = pl.pallas_call(
    _tc2_body,
    out_shape=jax.ShapeDtypeStruct((NPAD_P, 128), jnp.float32),
)

_tc3 = pl.pallas_call(
    _tc3_body,
    out_shape=jax.ShapeDtypeStruct((NP_P, 128), jnp.float32),
)


def kernel(x, edge_index_curr, W1, b1, W2, b2):
    zeros1d = jnp.zeros((NPAD,), jnp.float32)
    ones1d = jnp.ones((CHUNK,), jnp.float32)
    zeros2d = jnp.zeros((NPAD, D_HID), jnp.float32)

    w1b = jax.scipy.linalg.block_diag(*([W1] * PK))
    w2b = jax.scipy.linalg.block_diag(*([W2] * PK))
    expand = jnp.kron(jnp.eye(PK, dtype=jnp.float32),
                      jnp.ones((1, D_HID), jnp.float32))
    b1t = jnp.tile(b1, PK).reshape(1, 128)
    b2t = jnp.tile(b2, PK).reshape(1, 128)
    xp = x.reshape(NP_P, PK * D_IN)

    degp = _deg_kernel(edge_index_curr, ones1d, zeros1d)
    h1 = _tcmm(xp, w1b)
    hs1, dis = _tcscale(degp.reshape(NC, NPAD_P, PK), expand, h1)
    aggp1 = _agg_kernel(edge_index_curr, hs1.reshape(NPAD, D_HID), zeros2d)
    hs2 = _tc2(aggp1.reshape(NC, NPAD_P, 128), hs1, dis, b1t, w2b)
    aggp2 = _agg_kernel(edge_index_curr, hs2.reshape(NPAD, D_HID), zeros2d)
    out_p = _tc3(aggp2.reshape(NC, NPAD_P, 128), hs2, dis, b2t)
    return out_p.reshape(N, D_HID)


# speedup vs baseline: 1.0082x; 1.0082x over previous
import functools

import jax
import jax.numpy as jnp
from jax import lax
from jax.experimental import pallas as pl
from jax.experimental.pallas import tpu as pltpu
from jax.experimental.pallas import tpu_sc as plsc

N = 10000
NPAD = 10240
E = 320000
D_IN = 128
D_HID = 16
PK = 128 // D_HID
NP_P = N // PK
NPAD_P = NPAD // PK

NC = 2
NS = 16
NW = NC * NS
EPW = E // NW
CHUNKS = 10
CHUNK = EPW // CHUNKS
NBUF = 4
NODES_PER_S = NPAD // NS

_mesh = plsc.VectorSubcoreMesh(core_axis_name="c", subcore_axis_name="s")


@functools.partial(
    pl.kernel,
    out_type=jax.ShapeDtypeStruct((NC, NPAD), jnp.float32),
    mesh=_mesh,
    scratch_types=[
        [pltpu.VMEM((CHUNK,), jnp.int32) for _ in range(CHUNKS)],
        pltpu.VMEM((CHUNK,), jnp.float32),
        pltpu.VMEM_SHARED((NPAD,), jnp.float32),
        pltpu.SemaphoreType.DMA,
        pltpu.SemaphoreType.DMA,
    ],
    compiler_params=pltpu.CompilerParams(use_tc_tiling_on_sc=False),
)
def _deg_kernel(ei_hbm, ones_hbm, zeros_hbm, out_hbm, dst_vs, ones_v, acc_s,
                isem, ssem):
    c = lax.axis_index("c")
    s = lax.axis_index("s")
    wid = s * NC + c
    sl = pl.ds(s * NODES_PER_S, NODES_PER_S)
    e0 = wid * EPW
    cps = [pltpu.async_copy(ei_hbm.at[1, pl.ds(e0 + k * CHUNK, CHUNK)],
                            dst_vs[k], isem) for k in range(CHUNKS)]
    one_cp = pltpu.async_copy(ones_hbm, ones_v, isem)
    pltpu.sync_copy(zeros_hbm.at[sl], acc_s.at[sl])
    for cp in cps:
        cp.wait()
    one_cp.wait()
    plsc.subcore_barrier()
    scps = [pltpu.async_copy(ones_v, acc_s.at[dst_vs[k]], ssem, add=True)
            for k in range(CHUNKS)]
    for cp in scps:
        cp.wait()
    plsc.subcore_barrier()
    pltpu.sync_copy(acc_s.at[sl], out_hbm.at[c, sl])


@functools.partial(
    pl.kernel,
    out_type=jax.ShapeDtypeStruct((NC, NPAD, D_HID), jnp.float32),
    mesh=_mesh,
    scratch_types=[
        pltpu.VMEM((EPW,), jnp.int32),
        [pltpu.VMEM((CHUNK,), jnp.int32) for _ in range(CHUNKS)],
        [pltpu.VMEM((CHUNK, D_HID), jnp.float32) for _ in range(NBUF)],
        pltpu.VMEM_SHARED((NPAD, D_HID), jnp.float32),
        pltpu.SemaphoreType.DMA,
        [pltpu.SemaphoreType.DMA for _ in range(NBUF)],
        [pltpu.SemaphoreType.DMA for _ in range(NBUF)],
    ],
    compiler_params=pltpu.CompilerParams(use_tc_tiling_on_sc=False),
)
def _agg_kernel(ei_hbm, hs_hbm, zeros_hbm, out_hbm,
                src_all, dst_vs, rows_vs, acc_s, isem, gsems, ssems):
    c = lax.axis_index("c")
    s = lax.axis_index("s")
    wid = s * NC + c
    sl = pl.ds(s * NODES_PER_S, NODES_PER_S)
    e0 = wid * EPW

    def gather(k):
        return pltpu.async_copy(
            hs_hbm.at[src_all.at[pl.ds(k * CHUNK, CHUNK)]],
            rows_vs[k % NBUF], gsems[k % NBUF])

    src_cp = pltpu.async_copy(ei_hbm.at[0, pl.ds(e0, EPW)], src_all, isem)
    dst_cps = [pltpu.async_copy(ei_hbm.at[1, pl.ds(e0 + k * CHUNK, CHUNK)],
                                dst_vs[k], isem) for k in range(CHUNKS)]
    pltpu.sync_copy(zeros_hbm.at[sl], acc_s.at[sl])
    src_cp.wait()
    for cp in dst_cps:
        cp.wait()
    plsc.subcore_barrier()
    gcps = [None] * NBUF
    scps = [None] * NBUF
    for k in range(min(NBUF, CHUNKS)):
        gcps[k] = gather(k)
    for k in range(CHUNKS):
        b = k % NBUF
        gcps[b].wait()
        scps[b] = pltpu.async_copy(rows_vs[b], acc_s.at[dst_vs[k]], ssems[b],
                                   add=True)
        if k + NBUF < CHUNKS:
            scps[b].wait()
            gcps[b] = gather(k + NBUF)
    for k in range(max(0, CHUNKS - NBUF), CHUNKS):
        scps[k % NBUF].wait()
    plsc.subcore_barrier()
    pltpu.sync_copy(acc_s.at[sl], out_hbm.at[c, sl])


def _tcmm_body(xp_ref, w1b_ref, h_ref):
    h_ref[:NP_P, :] = jnp.dot(xp_ref[...], w1b_ref[...],
                              preferred_element_type=jnp.float32)
    h_ref[NP_P:, :] = jnp.zeros((NPAD_P - NP_P, 128), jnp.float32)


def _tcscale_body(degp_ref, exp_ref, h_ref, hs_ref, dis_ref):
    g = lax.rsqrt(1.0 + degp_ref[0] + degp_ref[1])
    dis = jnp.dot(g, exp_ref[...], preferred_element_type=jnp.float32)
    hs_ref[...] = dis * h_ref[...]
    dis_ref[...] = dis


def _tc2_body(aggp_ref, hs1_ref, dis_ref, b1_ref, w2b_ref, hs2_ref):
    dis = dis_ref[...]
    t = dis * (aggp_ref[0] + aggp_ref[1] + hs1_ref[...])
    t = jnp.maximum(t + b1_ref[...], 0.0)
    u = jnp.dot(t, w2b_ref[...], preferred_element_type=jnp.float32)
    hs2_ref[:NP_P, :] = dis[:NP_P, :] * u[:NP_P, :]
    hs2_ref[NP_P:, :] = jnp.zeros((NPAD_P - NP_P, 128), jnp.float32)


def _tc3_body(aggp_ref, hs2_ref, dis_ref, b2_ref, out_ref):
    v = dis_ref[:NP_P, :] * (aggp_ref[0, :NP_P, :] + aggp_ref[1, :NP_P, :]
                             + hs2_ref[:NP_P, :]) + b2_ref[...]
    out_ref[...] = jax.nn.sigmoid(v)


_tcmm = pl.pallas_call(
    _tcmm_body,
    out_shape=jax.ShapeDtypeStruct((NPAD_P, 128), jnp.float32),
)

_tcscale = pl.pallas_call(
    _tcscale_body,
    out_shape=[jax.ShapeDtypeStruct((NPAD_P, 128), jnp.float32),
               jax.ShapeDtypeStruct((NPAD_P, 128), jnp.float32)],
)

_tc2 = pl.pallas_call(
    _tc2_body,
    out_shape=jax.ShapeDtypeStruct((NPAD_P, 128), jnp.float32),
)

_tc3 = pl.pallas_call(
    _tc3_body,
    out_shape=jax.ShapeDtypeStruct((NP_P, 128), jnp.float32),
)


def kernel(x, edge_index_curr, W1, b1, W2, b2):
    zeros1d = jnp.zeros((NPAD,), jnp.float32)
    ones1d = jnp.ones((CHUNK,), jnp.float32)
    zeros2d = jnp.zeros((NPAD, D_HID), jnp.float32)

    w1b = jax.scipy.linalg.block_diag(*([W1] * PK))
    w2b = jax.scipy.linalg.block_diag(*([W2] * PK))
    expand = jnp.kron(jnp.eye(PK, dtype=jnp.float32),
                      jnp.ones((1, D_HID), jnp.float32))
    b1t = jnp.tile(b1, PK).reshape(1, 128)
    b2t = jnp.tile(b2, PK).reshape(1, 128)
    xp = x.reshape(NP_P, PK * D_IN)

    degp = _deg_kernel(edge_index_curr, ones1d, zeros1d)
    h1 = _tcmm(xp, w1b)
    hs1, dis = _tcscale(degp.reshape(NC, NPAD_P, PK), expand, h1)
    aggp1 = _agg_kernel(edge_index_curr, hs1.reshape(NPAD, D_HID), zeros2d)
    hs2 = _tc2(aggp1.reshape(NC, NPAD_P, 128), hs1, dis, b1t, w2b)
    aggp2 = _agg_kernel(edge_index_curr, hs2.reshape(NPAD, D_HID), zeros2d)
    out_p = _tc3(aggp2.reshape(NC, NPAD_P, 128), hs2, dis, b2t)
    return out_p.reshape(N, D_HID)
